# baseline (speedup 1.0000x reference)
import functools

import jax
import jax.numpy as jnp
from jax import lax
from jax.experimental import pallas as pl
from jax.experimental.pallas import tpu as pltpu
from jax.experimental.pallas import tpu_sc as plsc

E = 8
TOP_K = 2
N_GROUP = 4
TOPK_GROUP = 2
D = 1024
I = 1408
SCALE = 1.0
T = 2048
P = T * TOP_K
TM = 512
TG = 512
NT = P // TG
W = 16
ICS = 256
Is = 2 * I
NICS = Is // ICS
NW = 32
CH = 32

_bf16 = jnp.bfloat16


def _mm_t(a, b, prefer=jnp.float32):
    return jax.lax.dot_general(
        a, b, (((1,), (1,)), ((), ())), preferred_element_type=prefer
    )


def _router_body(x_ref, gw_ref, eids_ref, ws_ref):
    x = x_ref[...]
    s = jax.nn.softmax(_mm_t(x, gw_ref[...]), axis=-1)
    gcols = [
        jnp.maximum(s[:, 2 * j : 2 * j + 1], s[:, 2 * j + 1 : 2 * j + 2])
        for j in range(N_GROUP)
    ]
    G = jnp.concatenate(gcols, axis=1)
    jidx = jax.lax.broadcasted_iota(jnp.int32, G.shape, 1)
    m1 = jnp.max(G, axis=1, keepdims=True)
    i1 = jnp.min(jnp.where(G == m1, jidx, N_GROUP), axis=1, keepdims=True)
    G2 = jnp.where(jidx == i1, -jnp.inf, G)
    m2 = jnp.max(G2, axis=1, keepdims=True)
    i2 = jnp.min(jnp.where(G2 == m2, jidx, N_GROUP), axis=1, keepdims=True)
    gmask = jnp.where((jidx == i1) | (jidx == i2), 1.0, 0.0)
    smcols = [gmask[:, j // 2 : j // 2 + 1] for j in range(E)]
    smask = jnp.concatenate(smcols, axis=1)
    ms = jnp.where(smask > 0, s, 0.0)
    eidx = jax.lax.broadcasted_iota(jnp.int32, ms.shape, 1)
    w1 = jnp.max(ms, axis=1, keepdims=True)
    e1 = jnp.min(jnp.where(ms == w1, eidx, E), axis=1, keepdims=True)
    ms2 = jnp.where(eidx == e1, -jnp.inf, ms)
    w2 = jnp.max(ms2, axis=1, keepdims=True)
    e2 = jnp.min(jnp.where(ms2 == w2, eidx, E), axis=1, keepdims=True)
    tot = w1 + w2 + 1e-20
    eids_ref[...] = jnp.concatenate([e1, e2], axis=1)
    ws_ref[...] = jnp.concatenate([w1 / tot, w2 / tot], axis=1)


def _router(x, gate_w):
    return pl.pallas_call(
        _router_body,
        grid=(T // TM,),
        in_specs=[
            pl.BlockSpec((TM, D), lambda t: (t, 0)),
            pl.BlockSpec((E, D), lambda t: (0, 0)),
        ],
        out_specs=[
            pl.BlockSpec((TM, TOP_K), lambda t: (t, 0)),
            pl.BlockSpec((TM, TOP_K), lambda t: (t, 0)),
        ],
        out_shape=[
            jax.ShapeDtypeStruct((T, TOP_K), jnp.int32),
            jax.ShapeDtypeStruct((T, TOP_K), jnp.float32),
        ],
    )(x, gate_w)


def _sc_gather(table, idx):
    B = idx.shape[0]
    bw = B // NW
    mesh = plsc.VectorSubcoreMesh(core_axis_name="c", subcore_axis_name="s")

    @functools.partial(
        pl.kernel,
        mesh=mesh,
        out_type=jax.ShapeDtypeStruct((B, D), jnp.float32),
        scratch_types=[
            pltpu.VMEM((CH,), jnp.int32),
            pltpu.VMEM((CH,), jnp.int32),
            pltpu.VMEM((CH, D), jnp.float32),
            pltpu.VMEM((CH, D), jnp.float32),
            pltpu.SemaphoreType.DMA,
            pltpu.SemaphoreType.DMA,
        ],
    )
    def k(table_hbm, idx_hbm, out_hbm, idx0, idx1, rows0, rows1, sem0, sem1):
        wid = lax.axis_index("s") * 2 + lax.axis_index("c")
        base = wid * bw
        idxb = (idx0, idx1)
        rowsb = (rows0, rows1)
        semb = (sem0, sem1)
        nch = bw // CH
        cps = [None, None]
        for c in range(nch):
            b = c % 2
            if cps[b] is not None:
                cps[b].wait()
                pltpu.sync_copy(
                    rowsb[b], out_hbm.at[pl.ds(base + (c - 2) * CH, CH)]
                )
            pltpu.sync_copy(idx_hbm.at[pl.ds(base + c * CH, CH)], idxb[b])
            cps[b] = pltpu.async_copy(table_hbm.at[idxb[b]], rowsb[b], semb[b])
        for c in range(nch - 2, nch):
            b = c % 2
            cps[b].wait()
            pltpu.sync_copy(rowsb[b], out_hbm.at[pl.ds(base + c * CH, CH)])

    return k(table, idx)


def _gmm_body(tile_r, exp_r, st_r, en_r, fst_r, xs_ref, wg_ref, wu_ref,
              wd_ref, h_ref):
    j = pl.program_id(0)
    xb = xs_ref[...]
    g = _mm_t(xb, wg_ref[0])
    u = _mm_t(xb, wu_ref[0])
    act = jax.nn.silu(g) * u
    h = _mm_t(act, wd_ref[0])
    r = tile_r[j] * TG + jax.lax.broadcasted_iota(jnp.int32, (TG, 1), 0)
    m = (r >= st_r[j]) & (r < en_r[j])
    hm = jnp.where(m, h, 0.0)

    @pl.when(fst_r[j] == 1)
    def _():
        h_ref[...] = hm

    @pl.when(fst_r[j] == 0)
    def _():
        h_ref[...] += hm


def _gmm(tile_of, exp_of, start_of, end_of, first_of, xs, w_gate_up, w_down):
    grid_spec = pltpu.PrefetchScalarGridSpec(
        num_scalar_prefetch=5,
        grid=(W,),
        in_specs=[
            pl.BlockSpec((TG, D), lambda j, tr, er, sr, nr, fr: (tr[j], 0)),
            pl.BlockSpec((1, I, D), lambda j, tr, er, sr, nr, fr: (er[j], 0, 0)),
            pl.BlockSpec((1, I, D), lambda j, tr, er, sr, nr, fr: (er[j], 1, 0)),
            pl.BlockSpec((1, D, I), lambda j, tr, er, sr, nr, fr: (er[j], 0, 0)),
        ],
        out_specs=pl.BlockSpec((TG, D), lambda j, tr, er, sr, nr, fr: (tr[j], 0)),
        scratch_shapes=[],
    )
    return pl.pallas_call(
        _gmm_body,
        grid_spec=grid_spec,
        out_shape=jax.ShapeDtypeStruct((P, D), jnp.float32),
        compiler_params=pltpu.CompilerParams(
            dimension_semantics=("arbitrary",),
        ),
    )(tile_of, exp_of, start_of, end_of, first_of, xs, w_gate_up, w_gate_up,
      w_down)


def _shared_body(x_ref, sg_ref, su_ref, sd_ref, out_ref):
    x = x_ref[...]
    g = _mm_t(x, sg_ref[...])
    u = _mm_t(x, su_ref[...])
    act = jax.nn.silu(g) * u
    out_ref[...] = _mm_t(act, sd_ref[...])


def _shared_ffn(x, shared_w_gate_up, shared_w_down):
    return pl.pallas_call(
        _shared_body,
        grid=(T // TM,),
        in_specs=[
            pl.BlockSpec((TM, D), lambda t: (t, 0)),
            pl.BlockSpec((Is, D), lambda t: (0, 0)),
            pl.BlockSpec((Is, D), lambda t: (1, 0)),
            pl.BlockSpec((D, Is), lambda t: (0, 0)),
        ],
        out_specs=pl.BlockSpec((TM, D), lambda t: (t, 0)),
        out_shape=jax.ShapeDtypeStruct((T, D), jnp.float32),
        compiler_params=pltpu.CompilerParams(
            dimension_semantics=("arbitrary",),
        ),
    )(x, shared_w_gate_up, shared_w_gate_up, shared_w_down)


def _combine_body(sh_ref, a_ref, b_ref, ws_ref, out_ref):
    t = pl.program_id(0)
    rows = pl.ds(t * TM, TM)
    w1 = ws_ref[rows, 0:1]
    w2 = ws_ref[rows, 1:2]
    out_ref[...] = sh_ref[...] + jnp.float32(SCALE) * (
        w1 * a_ref[...] + w2 * b_ref[...]
    )


def _combine(sh, ab, ws):
    return pl.pallas_call(
        _combine_body,
        grid=(T // TM,),
        in_specs=[
            pl.BlockSpec((TM, D), lambda t: (t, 0)),
            pl.BlockSpec((TM, D), lambda t: (t, 0)),
            pl.BlockSpec((TM, D), lambda t: (T // TM + t, 0)),
            pl.BlockSpec((T, TOP_K), lambda t: (0, 0)),
        ],
        out_specs=pl.BlockSpec((TM, D), lambda t: (t, 0)),
        out_shape=jax.ShapeDtypeStruct((T, D), jnp.float32),
    )(sh, ab, ab, ws)


@jax.jit
def _moe(x, gate_w, w_gate_up, w_down, shared_w_gate_up, shared_w_down):
    eids, ws = _router(x, gate_w)
    sh = _shared_ffn(x, shared_w_gate_up, shared_w_down)

    pe = eids.reshape(-1)
    oh = (pe[:, None] == jnp.arange(E, dtype=jnp.int32)[None, :]).astype(
        jnp.int32
    )
    ranks = jnp.cumsum(oh, axis=0) - oh
    counts = jnp.sum(oh, axis=0)
    offs = jnp.concatenate(
        [jnp.zeros((1,), jnp.int32), jnp.cumsum(counts).astype(jnp.int32)]
    )
    pos = jnp.sum((ranks + offs[None, :E]) * oh, axis=1)
    tok = (
        jnp.zeros((P,), jnp.int32)
        .at[pos]
        .set(jnp.arange(P, dtype=jnp.int32) // TOP_K, unique_indices=True)
    )
    tile_lo = jnp.arange(NT, dtype=jnp.int32)[:, None] * TG
    ov = (offs[None, :E] < tile_lo + TG) & (offs[None, 1:] > tile_lo)
    nz = jnp.nonzero(ov.reshape(-1), size=W, fill_value=0)[0].astype(jnp.int32)
    nv = jnp.sum(ov.astype(jnp.int32))
    jj = jnp.arange(W, dtype=jnp.int32)
    valid = jj < nv
    tile_of = jnp.where(valid, nz // E, NT - 1).astype(jnp.int32)
    exp_of = jnp.where(valid, nz % E, 0).astype(jnp.int32)
    start_of = jnp.where(valid, jnp.maximum(offs[exp_of], tile_of * TG), 0)
    end_of = jnp.where(
        valid, jnp.minimum(offs[exp_of + 1], (tile_of + 1) * TG), 0
    )
    prev_tile = jnp.concatenate([jnp.full((1,), -1, jnp.int32), tile_of[:-1]])
    first_of = (tile_of != prev_tile).astype(jnp.int32)

    xs = _sc_gather(x, tok)
    h = _gmm(tile_of, exp_of, start_of, end_of, first_of, xs, w_gate_up,
             w_down)
    cidx = jnp.concatenate([pos[0::TOP_K], pos[1::TOP_K]])
    ab = _sc_gather(h, cidx)
    return _combine(sh, ab, ws)


def kernel(hidden_states, gate_w, w_gate_up, w_down, shared_w_gate_up,
           shared_w_down):
    return _moe(hidden_states, gate_w, w_gate_up, w_down, shared_w_gate_up,
                shared_w_down)


# speedup vs baseline: 1.0163x; 1.0163x over previous
import functools

import jax
import jax.numpy as jnp
from jax import lax
from jax.experimental import pallas as pl
from jax.experimental.pallas import tpu as pltpu
from jax.experimental.pallas import tpu_sc as plsc

E = 8
TOP_K = 2
N_GROUP = 4
TOPK_GROUP = 2
D = 1024
I = 1408
SCALE = 1.0
T = 2048
P = T * TOP_K
TM = 512
TG = 512
NT = P // TG
W = 16
ICS = 256
Is = 2 * I
NICS = Is // ICS
NW = 32
CH = 64

_bf16 = jnp.bfloat16


def _mm_t(a, b, prefer=jnp.float32):
    return jax.lax.dot_general(
        a, b, (((1,), (1,)), ((), ())), preferred_element_type=prefer
    )


def _router_body(x_ref, gw_ref, eids_ref, ws_ref):
    x = x_ref[...]
    s = jax.nn.softmax(_mm_t(x, gw_ref[...]), axis=-1)
    gcols = [
        jnp.maximum(s[:, 2 * j : 2 * j + 1], s[:, 2 * j + 1 : 2 * j + 2])
        for j in range(N_GROUP)
    ]
    G = jnp.concatenate(gcols, axis=1)
    jidx = jax.lax.broadcasted_iota(jnp.int32, G.shape, 1)
    m1 = jnp.max(G, axis=1, keepdims=True)
    i1 = jnp.min(jnp.where(G == m1, jidx, N_GROUP), axis=1, keepdims=True)
    G2 = jnp.where(jidx == i1, -jnp.inf, G)
    m2 = jnp.max(G2, axis=1, keepdims=True)
    i2 = jnp.min(jnp.where(G2 == m2, jidx, N_GROUP), axis=1, keepdims=True)
    gmask = jnp.where((jidx == i1) | (jidx == i2), 1.0, 0.0)
    smcols = [gmask[:, j // 2 : j // 2 + 1] for j in range(E)]
    smask = jnp.concatenate(smcols, axis=1)
    ms = jnp.where(smask > 0, s, 0.0)
    eidx = jax.lax.broadcasted_iota(jnp.int32, ms.shape, 1)
    w1 = jnp.max(ms, axis=1, keepdims=True)
    e1 = jnp.min(jnp.where(ms == w1, eidx, E), axis=1, keepdims=True)
    ms2 = jnp.where(eidx == e1, -jnp.inf, ms)
    w2 = jnp.max(ms2, axis=1, keepdims=True)
    e2 = jnp.min(jnp.where(ms2 == w2, eidx, E), axis=1, keepdims=True)
    tot = w1 + w2 + 1e-20
    eids_ref[...] = jnp.concatenate([e1, e2], axis=1)
    ws_ref[...] = jnp.concatenate([w1 / tot, w2 / tot], axis=1)


def _router(x, gate_w):
    return pl.pallas_call(
        _router_body,
        grid=(T // TM,),
        in_specs=[
            pl.BlockSpec((TM, D), lambda t: (t, 0)),
            pl.BlockSpec((E, D), lambda t: (0, 0)),
        ],
        out_specs=[
            pl.BlockSpec((TM, TOP_K), lambda t: (t, 0)),
            pl.BlockSpec((TM, TOP_K), lambda t: (t, 0)),
        ],
        out_shape=[
            jax.ShapeDtypeStruct((T, TOP_K), jnp.int32),
            jax.ShapeDtypeStruct((T, TOP_K), jnp.float32),
        ],
    )(x, gate_w)


def _sc_gather(table, idx):
    B = idx.shape[0]
    bw = B // NW
    mesh = plsc.VectorSubcoreMesh(core_axis_name="c", subcore_axis_name="s")

    @functools.partial(
        pl.kernel,
        mesh=mesh,
        out_type=jax.ShapeDtypeStruct((B, D), jnp.float32),
        scratch_types=[
            pltpu.VMEM((CH,), jnp.int32),
            pltpu.VMEM((CH, D), jnp.float32),
            pltpu.SemaphoreType.DMA,
        ],
    )
    def k(table_hbm, idx_hbm, out_hbm, idx_v, rows_v, sem):
        wid = lax.axis_index("s") * 2 + lax.axis_index("c")
        base = wid * bw
        for c in range(bw // CH):
            off = base + c * CH
            pltpu.sync_copy(idx_hbm.at[pl.ds(off, CH)], idx_v)
            pltpu.async_copy(table_hbm.at[idx_v], rows_v, sem).wait()
            pltpu.sync_copy(rows_v, out_hbm.at[pl.ds(off, CH)])

    return k(table, idx)


def _gmm_body(tile_r, exp_r, st_r, en_r, fst_r, xs_ref, wg_ref, wu_ref,
              wd_ref, h_ref):
    j = pl.program_id(0)
    xb = xs_ref[...]
    g = _mm_t(xb, wg_ref[0])
    u = _mm_t(xb, wu_ref[0])
    act = jax.nn.silu(g) * u
    h = _mm_t(act, wd_ref[0])
    r = tile_r[j] * TG + jax.lax.broadcasted_iota(jnp.int32, (TG, 1), 0)
    m = (r >= st_r[j]) & (r < en_r[j])
    hm = jnp.where(m, h, 0.0)

    @pl.when(fst_r[j] == 1)
    def _():
        h_ref[...] = hm

    @pl.when(fst_r[j] == 0)
    def _():
        h_ref[...] += hm


def _gmm(tile_of, exp_of, start_of, end_of, first_of, xs, w_gate_up, w_down):
    grid_spec = pltpu.PrefetchScalarGridSpec(
        num_scalar_prefetch=5,
        grid=(W,),
        in_specs=[
            pl.BlockSpec((TG, D), lambda j, tr, er, sr, nr, fr: (tr[j], 0)),
            pl.BlockSpec((1, I, D), lambda j, tr, er, sr, nr, fr: (er[j], 0, 0)),
            pl.BlockSpec((1, I, D), lambda j, tr, er, sr, nr, fr: (er[j], 1, 0)),
            pl.BlockSpec((1, D, I), lambda j, tr, er, sr, nr, fr: (er[j], 0, 0)),
        ],
        out_specs=pl.BlockSpec((TG, D), lambda j, tr, er, sr, nr, fr: (tr[j], 0)),
        scratch_shapes=[],
    )
    return pl.pallas_call(
        _gmm_body,
        grid_spec=grid_spec,
        out_shape=jax.ShapeDtypeStruct((P, D), jnp.float32),
        compiler_params=pltpu.CompilerParams(
            dimension_semantics=("arbitrary",),
        ),
    )(tile_of, exp_of, start_of, end_of, first_of, xs, w_gate_up, w_gate_up,
      w_down)


def _shared_body(x_ref, sg_ref, su_ref, sd_ref, out_ref):
    x = x_ref[...]
    g = _mm_t(x, sg_ref[...])
    u = _mm_t(x, su_ref[...])
    act = jax.nn.silu(g) * u
    out_ref[...] = _mm_t(act, sd_ref[...])


def _shared_ffn(x, shared_w_gate_up, shared_w_down):
    return pl.pallas_call(
        _shared_body,
        grid=(T // TM,),
        in_specs=[
            pl.BlockSpec((TM, D), lambda t: (t, 0)),
            pl.BlockSpec((Is, D), lambda t: (0, 0)),
            pl.BlockSpec((Is, D), lambda t: (1, 0)),
            pl.BlockSpec((D, Is), lambda t: (0, 0)),
        ],
        out_specs=pl.BlockSpec((TM, D), lambda t: (t, 0)),
        out_shape=jax.ShapeDtypeStruct((T, D), jnp.float32),
        compiler_params=pltpu.CompilerParams(
            dimension_semantics=("arbitrary",),
        ),
    )(x, shared_w_gate_up, shared_w_gate_up, shared_w_down)


def _combine_body(sh_ref, a_ref, b_ref, ws_ref, out_ref):
    t = pl.program_id(0)
    rows = pl.ds(t * TM, TM)
    w1 = ws_ref[rows, 0:1]
    w2 = ws_ref[rows, 1:2]
    out_ref[...] = sh_ref[...] + jnp.float32(SCALE) * (
        w1 * a_ref[...] + w2 * b_ref[...]
    )


def _combine(sh, ab, ws):
    return pl.pallas_call(
        _combine_body,
        grid=(T // TM,),
        in_specs=[
            pl.BlockSpec((TM, D), lambda t: (t, 0)),
            pl.BlockSpec((TM, D), lambda t: (t, 0)),
            pl.BlockSpec((TM, D), lambda t: (T // TM + t, 0)),
            pl.BlockSpec((T, TOP_K), lambda t: (0, 0)),
        ],
        out_specs=pl.BlockSpec((TM, D), lambda t: (t, 0)),
        out_shape=jax.ShapeDtypeStruct((T, D), jnp.float32),
    )(sh, ab, ab, ws)


@jax.jit
def _moe(x, gate_w, w_gate_up, w_down, shared_w_gate_up, shared_w_down):
    eids, ws = _router(x, gate_w)
    sh = _shared_ffn(x, shared_w_gate_up, shared_w_down)

    pe = eids.reshape(-1)
    oh = (pe[:, None] == jnp.arange(E, dtype=jnp.int32)[None, :]).astype(
        jnp.int32
    )
    ranks = jnp.cumsum(oh, axis=0) - oh
    counts = jnp.sum(oh, axis=0)
    offs = jnp.concatenate(
        [jnp.zeros((1,), jnp.int32), jnp.cumsum(counts).astype(jnp.int32)]
    )
    pos = jnp.sum((ranks + offs[None, :E]) * oh, axis=1)
    tok = (
        jnp.zeros((P,), jnp.int32)
        .at[pos]
        .set(jnp.arange(P, dtype=jnp.int32) // TOP_K, unique_indices=True)
    )
    tile_lo = jnp.arange(NT, dtype=jnp.int32)[:, None] * TG
    ov = (offs[None, :E] < tile_lo + TG) & (offs[None, 1:] > tile_lo)
    nz = jnp.nonzero(ov.reshape(-1), size=W, fill_value=0)[0].astype(jnp.int32)
    nv = jnp.sum(ov.astype(jnp.int32))
    jj = jnp.arange(W, dtype=jnp.int32)
    valid = jj < nv
    tile_of = jnp.where(valid, nz // E, NT - 1).astype(jnp.int32)
    exp_of = jnp.where(valid, nz % E, 0).astype(jnp.int32)
    start_of = jnp.where(valid, jnp.maximum(offs[exp_of], tile_of * TG), 0)
    end_of = jnp.where(
        valid, jnp.minimum(offs[exp_of + 1], (tile_of + 1) * TG), 0
    )
    prev_tile = jnp.concatenate([jnp.full((1,), -1, jnp.int32), tile_of[:-1]])
    first_of = (tile_of != prev_tile).astype(jnp.int32)

    xs = _sc_gather(x, tok)
    h = _gmm(tile_of, exp_of, start_of, end_of, first_of, xs, w_gate_up,
             w_down)
    cidx = jnp.concatenate([pos[0::TOP_K], pos[1::TOP_K]])
    ab = _sc_gather(h, cidx)
    return _combine(sh, ab, ws)


def kernel(hidden_states, gate_w, w_gate_up, w_down, shared_w_gate_up,
           shared_w_down):
    return _moe(hidden_states, gate_w, w_gate_up, w_down, shared_w_gate_up,
                shared_w_down)
